# baseline (speedup 1.0000x reference)
import jax
import jax.numpy as jnp
from jax.experimental import pallas as pl
from jax.experimental.pallas import tpu as pltpu

B, Qq, D = 32, 32, 768
N, Q = 2048, 32
K = 9
KPAD = 16
BN = 128
NB = N // BN
CHUNK = 512
ALPHA = 0.8


def _retrieve_kernel(qf_ref, qt_ref, mp_ref, m_ref, vals_ref, idx_ref,
                     q2n, qfn, r1acc, sim_a, sim_b):
    i = pl.program_id(0)

    @pl.when(i == 0)
    def _prologue():
        x = qt_ref[...]
        nrm = jnp.sqrt(jnp.sum(x * x, axis=1, keepdims=True))
        xn = x / jnp.maximum(nrm, 1e-12)
        q2n[...] = xn.T.astype(jnp.bfloat16)
        y = qf_ref[...]
        n2 = jnp.sqrt(jnp.sum(y * y, axis=1, keepdims=True))
        qfn[...] = y / jnp.maximum(n2, 1e-12)

    q2 = q2n[...]
    H = BN * Q // CHUNK
    bufs = [sim_a, sim_b]
    r1_parts = []

    def _reduce(buf):
        return jnp.max(buf[...].reshape(CHUNK // Q, Q, B * Qq), axis=1)

    for h in range(H):
        mh = m_ref[pl.ds(h * CHUNK, CHUNK), :].astype(jnp.bfloat16)
        bufs[h % 2][...] = jax.lax.dot_general(
            mh, q2, (((1,), (0,)), ((), ())),
            preferred_element_type=jnp.float32,
            precision=jax.lax.Precision.DEFAULT)
        if h >= 1:
            r1_parts.append(_reduce(bufs[(h - 1) % 2]))
    r1_parts.append(_reduce(bufs[(H - 1) % 2]))
    r1 = jnp.concatenate(r1_parts, axis=0)
    r1acc[pl.ds(i * BN, BN), :] = r1

    @pl.when(i == NB - 1)
    def _topk():
        r1t = r1acc[...].T
        local = jnp.max(r1t.reshape(B, Qq, N), axis=1)
        scores = jax.lax.dot_general(
            qfn[...], mp_ref[...], (((1,), (1,)), ((), ())),
            preferred_element_type=jnp.float32,
            precision=jax.lax.Precision.DEFAULT)
        f = ALPHA * scores + (1.0 - ALPHA) * local
        col = jax.lax.broadcasted_iota(jnp.int32, (B, N), 1)
        kcol = jax.lax.broadcasted_iota(jnp.int32, (B, KPAD), 1)
        vacc = jnp.zeros((B, KPAD), jnp.float32)
        iacc = jnp.zeros((B, KPAD), jnp.int32)
        for k in range(K):
            m = jnp.max(f, axis=1, keepdims=True)
            am = jnp.min(jnp.where(f == m, col, N), axis=1, keepdims=True)
            vacc = jnp.where(kcol == k, m, vacc)
            iacc = jnp.where(kcol == k, am, iacc)
            f = jnp.where(col == am, jnp.float32(-jnp.inf), f)
        vals_ref[...] = vacc
        idx_ref[...] = iacc


def kernel(query_features, q_tokens, mem_pooled, mem_qtokens, top_k):
    qt2 = q_tokens.reshape(B * Qq, D)
    m2 = mem_qtokens.reshape(N * Q, D)
    vals, idx = pl.pallas_call(
        _retrieve_kernel,
        grid=(NB,),
        in_specs=[
            pl.BlockSpec((B, D), lambda i: (0, 0)),
            pl.BlockSpec((B * Qq, D), lambda i: (0, 0)),
            pl.BlockSpec((N, D), lambda i: (0, 0)),
            pl.BlockSpec((BN * Q, D), lambda i: (i, 0)),
        ],
        out_specs=[
            pl.BlockSpec((B, KPAD), lambda i: (0, 0)),
            pl.BlockSpec((B, KPAD), lambda i: (0, 0)),
        ],
        out_shape=[
            jax.ShapeDtypeStruct((B, KPAD), jnp.float32),
            jax.ShapeDtypeStruct((B, KPAD), jnp.int32),
        ],
        scratch_shapes=[
            pltpu.VMEM((D, B * Qq), jnp.bfloat16),
            pltpu.VMEM((B, D), jnp.float32),
            pltpu.VMEM((N, B * Qq), jnp.float32),
            pltpu.VMEM((CHUNK, B * Qq), jnp.float32),
            pltpu.VMEM((CHUNK, B * Qq), jnp.float32),
        ],
    )(query_features, qt2, mem_pooled, m2)
    return vals[:, :K], idx[:, :K]


# speedup vs baseline: 1.0082x; 1.0082x over previous
import jax
import jax.numpy as jnp
from jax.experimental import pallas as pl
from jax.experimental.pallas import tpu as pltpu

B, Qq, D = 32, 32, 768
N, Q = 2048, 32
K = 9
KPAD = 16
BN = 128
NB = N // BN
CHUNK = 512
ALPHA = 0.8


def _retrieve_kernel(qf_ref, qt_ref, mp_ref, m_ref, vals_ref, idx_ref,
                     q2n, qfn, fused, sim_a, sim_b):
    i = pl.program_id(0)

    @pl.when(i == 0)
    def _prologue():
        x = qt_ref[...]
        nrm = jnp.sqrt(jnp.sum(x * x, axis=1, keepdims=True))
        xn = x / jnp.maximum(nrm, 1e-12)
        q2n[...] = xn.T.astype(jnp.bfloat16)
        y = qf_ref[...]
        n2 = jnp.sqrt(jnp.sum(y * y, axis=1, keepdims=True))
        qfn[...] = y / jnp.maximum(n2, 1e-12)

    q2 = q2n[...]
    H = BN * Q // CHUNK
    bufs = [sim_a, sim_b]
    r1_parts = []

    def _reduce(buf):
        return jnp.max(buf[...].reshape(CHUNK // Q, Q, B * Qq), axis=1)

    for h in range(H):
        mh = m_ref[pl.ds(h * CHUNK, CHUNK), :].astype(jnp.bfloat16)
        bufs[h % 2][...] = jax.lax.dot_general(
            mh, q2, (((1,), (0,)), ((), ())),
            preferred_element_type=jnp.float32,
            precision=jax.lax.Precision.DEFAULT)
        if h >= 1:
            r1_parts.append(_reduce(bufs[(h - 1) % 2]))
    r1_parts.append(_reduce(bufs[(H - 1) % 2]))
    r1 = jnp.concatenate(r1_parts, axis=0)
    r1t = r1.T
    local = jnp.max(r1t.reshape(B, Qq, BN), axis=1)
    scores = jax.lax.dot_general(
        qfn[...], mp_ref[...], (((1,), (1,)), ((), ())),
        preferred_element_type=jnp.float32,
        precision=jax.lax.Precision.DEFAULT)
    fused[:, pl.ds(i * BN, BN)] = ALPHA * scores + (1.0 - ALPHA) * local

    @pl.when(i == NB - 1)
    def _topk():
        f = fused[...]
        col = jax.lax.broadcasted_iota(jnp.int32, (B, N), 1)
        kcol = jax.lax.broadcasted_iota(jnp.int32, (B, KPAD), 1)
        vacc = jnp.zeros((B, KPAD), jnp.float32)
        iacc = jnp.zeros((B, KPAD), jnp.int32)
        for k in range(K):
            m = jnp.max(f, axis=1, keepdims=True)
            am = jnp.min(jnp.where(f == m, col, N), axis=1, keepdims=True)
            vacc = jnp.where(kcol == k, m, vacc)
            iacc = jnp.where(kcol == k, am, iacc)
            f = jnp.where(col == am, jnp.float32(-jnp.inf), f)
        vals_ref[...] = vacc
        idx_ref[...] = iacc


def kernel(query_features, q_tokens, mem_pooled, mem_qtokens, top_k):
    qt2 = q_tokens.reshape(B * Qq, D)
    m2 = mem_qtokens.reshape(N * Q, D)
    vals, idx = pl.pallas_call(
        _retrieve_kernel,
        grid=(NB,),
        in_specs=[
            pl.BlockSpec((B, D), lambda i: (0, 0)),
            pl.BlockSpec((B * Qq, D), lambda i: (0, 0)),
            pl.BlockSpec((BN, D), lambda i: (i, 0)),
            pl.BlockSpec((BN * Q, D), lambda i: (i, 0)),
        ],
        out_specs=[
            pl.BlockSpec((B, KPAD), lambda i: (0, 0)),
            pl.BlockSpec((B, KPAD), lambda i: (0, 0)),
        ],
        out_shape=[
            jax.ShapeDtypeStruct((B, KPAD), jnp.float32),
            jax.ShapeDtypeStruct((B, KPAD), jnp.int32),
        ],
        scratch_shapes=[
            pltpu.VMEM((D, B * Qq), jnp.bfloat16),
            pltpu.VMEM((B, D), jnp.float32),
            pltpu.VMEM((B, N), jnp.float32),
            pltpu.VMEM((CHUNK, B * Qq), jnp.float32),
            pltpu.VMEM((CHUNK, B * Qq), jnp.float32),
        ],
    )(query_features, qt2, mem_pooled, m2)
    return vals[:, :K], idx[:, :K]
